# baseline (speedup 1.0000x reference)
import jax
import jax.numpy as jnp
from jax import lax
from jax.experimental import pallas as pl
from jax.experimental.pallas import tpu as pltpu
from jax.experimental.pallas import tpu_sc as plsc

B = 16384
L = 50
EMB_DIM = 64
N_WORDS_TOTAL = 1000000
TOTAL = B * L
NW = 32
PER_W = TOTAL // NW
CHUNK = 512
IDXW = 128
ROWS_PER_CHUNK = CHUNK // IDXW
N_CHUNKS = PER_W // CHUNK
GROUPS = CHUNK // 16
UNROLL = 8
WPR = EMB_DIM // 2

BK = 2048
NBLK = 125
SPLIT = NBLK * BK
NROWS_LIN = 4 * SPLIT
IN_BLOCKS = (N_WORDS_TOTAL + BK - 1) // BK - 1


def _pack_bf16_pair(lo32, hi32):
    ul = lax.bitcast_convert_type(lo32, jnp.uint32)
    uh = lax.bitcast_convert_type(hi32, jnp.uint32)
    hm = jnp.uint32(0xFFFF0000)
    rl = (ul + jnp.uint32(0x7FFF) + ((ul >> 16) & jnp.uint32(1))) & hm
    rh = (uh + jnp.uint32(0x7FFF) + ((uh >> 16) & jnp.uint32(1))) & hm
    return lax.bitcast_convert_type(rl >> 16 | rh, jnp.int32)


def _tr_body(a_ref, b_ref, c_ref, d_ref, o_ref):
    parts = []
    for r in (a_ref, b_ref, c_ref, d_ref):
        v = r[...]
        w = _pack_bf16_pair(v[:WPR, :], v[WPR:, :])
        parts.append(w.T)
    o_ref[...] = jnp.concatenate(parts, axis=1)


def _transpose_pack(tt):
    def spec(j):
        return pl.BlockSpec(
            (EMB_DIM, BK),
            lambda k, j=j: (0, jnp.minimum(j * NBLK + k, IN_BLOCKS)))
    return pl.pallas_call(
        _tr_body,
        grid=(NBLK,),
        in_specs=[spec(0), spec(1), spec(2), spec(3)],
        out_specs=pl.BlockSpec((BK, 4 * WPR), lambda k: (k, 0)),
        out_shape=jax.ShapeDtypeStruct((SPLIT, 4 * WPR), jnp.int32),
    )(tt, tt, tt, tt)


def _sc_body(wi_hbm, ci_hbm, table_hbm, params_hbm, out_hbm,
             wi_v, ci_v, wr, cr, outb, pv, sems, osems):
    nc = 2
    wid = lax.axis_index("s") * nc + lax.axis_index("c")

    pltpu.sync_copy(params_hbm, pv)
    bias = pv[pl.ds(EMB_DIM, 16)]
    fw = [pv[pl.ds(k * 16, 16)] for k in range(EMB_DIM // 16)]
    mask15 = lax.iota(jnp.int32, 16) == 15
    himask = jnp.full((16,), -65536, jnp.int32)

    idx_row0 = wid * (PER_W // IDXW)
    out_base = wid * PER_W

    pltpu.sync_copy(wi_hbm.at[pl.ds(idx_row0, PER_W // IDXW)], wi_v)
    pltpu.sync_copy(ci_hbm.at[pl.ds(idx_row0, PER_W // IDXW)], ci_v)

    def start_gathers(c, buf):
        row = c * ROWS_PER_CHUNK
        for j in range(ROWS_PER_CHUNK):
            pltpu.async_copy(table_hbm.at[wi_v.at[row + j]],
                             wr.at[buf, pl.ds(j * IDXW, IDXW)], sems.at[buf])
            pltpu.async_copy(table_hbm.at[ci_v.at[row + j]],
                             cr.at[buf, pl.ds(j * IDXW, IDXW)], sems.at[buf])

    def wait_gathers(c, buf):
        row = c * ROWS_PER_CHUNK
        for j in range(ROWS_PER_CHUNK):
            pltpu.make_async_copy(table_hbm.at[wi_v.at[row + j]],
                                  wr.at[buf, pl.ds(j * IDXW, IDXW)],
                                  sems.at[buf]).wait()
            pltpu.make_async_copy(table_hbm.at[ci_v.at[row + j]],
                                  cr.at[buf, pl.ds(j * IDXW, IDXW)],
                                  sems.at[buf]).wait()

    def unpack(v):
        lo = plsc.bitcast(v << 16, jnp.float32)
        hi = plsc.bitcast(v & himask, jnp.float32)
        return lo, hi

    def compute_chunk(c, buf):
        @pl.when(c >= 2)
        def _():
            pltpu.make_async_copy(
                outb.at[buf, pl.ds(0, CHUNK)],
                out_hbm.at[pl.ds(out_base + (c - 2) * CHUNK, CHUNK)],
                osems.at[buf]).wait()

        def look_body(i, _):
            accs = []
            for u in range(UNROLL):
                ii = i * UNROLL + u
                w0lo, w0hi = unpack(wr[buf, ii, pl.ds(0, 16)])
                w1lo, w1hi = unpack(wr[buf, ii, pl.ds(16, 16)])
                c0lo, c0hi = unpack(cr[buf, ii, pl.ds(0, 16)])
                c1lo, c1hi = unpack(cr[buf, ii, pl.ds(16, 16)])
                acc = ((w0lo * c0lo) * fw[0] + (w1lo * c1lo) * fw[1]
                       + (w0hi * c0hi) * fw[2] + (w1hi * c1hi) * fw[3])
                accs.append(acc)
            cums = [plsc.cumsum(a) for a in accs]
            for u, cum in enumerate(cums):
                z = 1.0 / (1.0 + jnp.exp(-(cum + bias)))
                plsc.store_compressed(outb.at[buf, pl.ds(i * UNROLL + u, 16)],
                                      z, mask=mask15)
            return 0

        lax.fori_loop(0, CHUNK // UNROLL, look_body, 0)
        pltpu.async_copy(outb.at[buf, pl.ds(0, CHUNK)],
                         out_hbm.at[pl.ds(out_base + c * CHUNK, CHUNK)],
                         osems.at[buf])

    start_gathers(0, 0)

    def pair_body(m, _):
        c0 = m * 2
        start_gathers(c0 + 1, 1)
        wait_gathers(c0, 0)
        compute_chunk(c0, 0)

        @pl.when(c0 + 2 < N_CHUNKS)
        def _():
            start_gathers(c0 + 2, 0)

        wait_gathers(c0 + 1, 1)
        compute_chunk(c0 + 1, 1)
        return 0

    lax.fori_loop(0, N_CHUNKS // 2, pair_body, 0)

    for buf, c in ((0, N_CHUNKS - 2), (1, N_CHUNKS - 1)):
        pltpu.make_async_copy(
            outb.at[buf, pl.ds(0, CHUNK)],
            out_hbm.at[pl.ds(out_base + c * CHUNK, CHUNK)],
            osems.at[buf]).wait()


@jax.jit
def _run(wi2d, ci2d, table, params):
    mesh = plsc.VectorSubcoreMesh(core_axis_name="c", subcore_axis_name="s")
    kern = pl.kernel(
        _sc_body,
        out_type=jax.ShapeDtypeStruct((TOTAL,), jnp.float32),
        mesh=mesh,
        scratch_types=[
            pltpu.VMEM((PER_W // IDXW, IDXW), jnp.int32),
            pltpu.VMEM((PER_W // IDXW, IDXW), jnp.int32),
            pltpu.VMEM((2, CHUNK, WPR), jnp.int32),
            pltpu.VMEM((2, CHUNK, WPR), jnp.int32),
            pltpu.VMEM((2, CHUNK + 16), jnp.float32),
            pltpu.VMEM((EMB_DIM + 16,), jnp.float32),
            pltpu.SemaphoreType.DMA((2,)),
            pltpu.SemaphoreType.DMA((2,)),
        ],
        compiler_params=pltpu.CompilerParams(
            needs_layout_passes=False, use_tc_tiling_on_sc=False),
    )
    return kern(wi2d, ci2d, table, params)


def kernel(word_ids, context_ids, table, fc_w, fc_b):
    packed = _transpose_pack(table.T.astype(jnp.float32))
    table_lin = packed.reshape(NROWS_LIN, WPR)

    def remap(ids):
        i = ids.T.reshape(TOTAL // IDXW, IDXW).astype(jnp.int32)
        return 4 * (i % SPLIT) + i // SPLIT

    wi2d = remap(word_ids)
    ci2d = remap(context_ids)
    params = jnp.concatenate(
        [fc_w.reshape(EMB_DIM), jnp.broadcast_to(fc_b, (16,))]).astype(jnp.float32)
    out = _run(wi2d, ci2d, table_lin, params)
    return out.reshape(L, B).T.reshape(B, L, 1)


# speedup vs baseline: 1.0548x; 1.0548x over previous
import jax
import jax.numpy as jnp
from jax import lax
from jax.experimental import pallas as pl
from jax.experimental.pallas import tpu as pltpu
from jax.experimental.pallas import tpu_sc as plsc

B = 16384
L = 50
EMB_DIM = 64
N_WORDS_TOTAL = 1000000
TOTAL = B * L
NW = 32
PER_W = TOTAL // NW
CHUNK = 512
IDXW = 128
ROWS_PER_CHUNK = CHUNK // IDXW
N_CHUNKS = PER_W // CHUNK
GROUPS = CHUNK // 16
UNROLL = 8
WPR = EMB_DIM // 2

BK = 2048
NBLK = 125
SPLIT = NBLK * BK
NROWS_LIN = 4 * SPLIT
IN_BLOCKS = (N_WORDS_TOTAL + BK - 1) // BK - 1


def _pack_bf16_pair(lo32, hi32):
    ul = lax.bitcast_convert_type(lo32, jnp.uint32)
    uh = lax.bitcast_convert_type(hi32, jnp.uint32)
    hm = jnp.uint32(0xFFFF0000)
    rl = (ul + jnp.uint32(0x7FFF) + ((ul >> 16) & jnp.uint32(1))) & hm
    rh = (uh + jnp.uint32(0x7FFF) + ((uh >> 16) & jnp.uint32(1))) & hm
    return lax.bitcast_convert_type(rl >> 16 | rh, jnp.int32)


def _tr_body(a_ref, b_ref, c_ref, d_ref, o_ref):
    parts = []
    for r in (a_ref, b_ref, c_ref, d_ref):
        v = r[...]
        w = _pack_bf16_pair(v[:WPR, :], v[WPR:, :])
        parts.append(w.T)
    o_ref[...] = jnp.concatenate(parts, axis=1)


def _transpose_pack(tt):
    def spec(j):
        return pl.BlockSpec(
            (EMB_DIM, BK),
            lambda k, j=j: (0, jnp.minimum(j * NBLK + k, IN_BLOCKS)))
    return pl.pallas_call(
        _tr_body,
        grid=(NBLK,),
        in_specs=[spec(0), spec(1), spec(2), spec(3)],
        out_specs=pl.BlockSpec((BK, 4 * WPR), lambda k: (k, 0)),
        out_shape=jax.ShapeDtypeStruct((SPLIT, 4 * WPR), jnp.int32),
    )(tt, tt, tt, tt)


def _sc_body(wi_hbm, ci_hbm, table_hbm, params_hbm, out_hbm,
             wi_v, ci_v, wr, cr, outb, pv, sems, osems):
    nc = 2
    wid = lax.axis_index("s") * nc + lax.axis_index("c")

    pltpu.sync_copy(params_hbm, pv)
    bias = pv[pl.ds(EMB_DIM, 16)]
    fw = [pv[pl.ds(k * 16, 16)] for k in range(EMB_DIM // 16)]
    mask15 = lax.iota(jnp.int32, 16) == 15
    himask = jnp.full((16,), -65536, jnp.int32)

    idx_row0 = wid * (PER_W // IDXW)
    out_base = wid * PER_W

    pltpu.sync_copy(wi_hbm.at[pl.ds(idx_row0, PER_W // IDXW)], wi_v)
    pltpu.sync_copy(ci_hbm.at[pl.ds(idx_row0, PER_W // IDXW)], ci_v)

    def start_gathers(c, buf):
        row = c * ROWS_PER_CHUNK
        for j in range(ROWS_PER_CHUNK):
            pltpu.async_copy(table_hbm.at[wi_v.at[row + j]],
                             wr.at[buf, pl.ds(j * IDXW, IDXW)], sems.at[buf])
            pltpu.async_copy(table_hbm.at[ci_v.at[row + j]],
                             cr.at[buf, pl.ds(j * IDXW, IDXW)], sems.at[buf])

    def wait_gathers(c, buf):
        row = c * ROWS_PER_CHUNK
        for j in range(ROWS_PER_CHUNK):
            pltpu.make_async_copy(table_hbm.at[wi_v.at[row + j]],
                                  wr.at[buf, pl.ds(j * IDXW, IDXW)],
                                  sems.at[buf]).wait()
            pltpu.make_async_copy(table_hbm.at[ci_v.at[row + j]],
                                  cr.at[buf, pl.ds(j * IDXW, IDXW)],
                                  sems.at[buf]).wait()

    def unpack(v):
        lo = plsc.bitcast(v << 16, jnp.float32)
        hi = plsc.bitcast(v & himask, jnp.float32)
        return lo, hi

    def compute_chunk(c, buf):
        @pl.when(c >= 2)
        def _():
            pltpu.make_async_copy(
                outb.at[buf, pl.ds(0, CHUNK)],
                out_hbm.at[pl.ds(out_base + (c - 2) * CHUNK, CHUNK)],
                osems.at[buf]).wait()

        def look_body(i, _):
            accs = []
            for u in range(UNROLL):
                ii = i * UNROLL + u
                w0lo, w0hi = unpack(wr[buf, ii, pl.ds(0, 16)])
                w1lo, w1hi = unpack(wr[buf, ii, pl.ds(16, 16)])
                c0lo, c0hi = unpack(cr[buf, ii, pl.ds(0, 16)])
                c1lo, c1hi = unpack(cr[buf, ii, pl.ds(16, 16)])
                acc = ((w0lo * c0lo) * fw[0] + (w1lo * c1lo) * fw[1]
                       + (w0hi * c0hi) * fw[2] + (w1hi * c1hi) * fw[3])
                accs.append(acc)
            cums = [plsc.cumsum(a) for a in accs]
            for u, cum in enumerate(cums):
                plsc.store_compressed(outb.at[buf, pl.ds(i * UNROLL + u, 16)],
                                      cum, mask=mask15)
            return 0

        lax.fori_loop(0, CHUNK // UNROLL, look_body, 0)

        def sig_body(g, _):
            v = outb[buf, pl.ds(g * 16, 16)] + bias
            outb[buf, pl.ds(g * 16, 16)] = 1.0 / (1.0 + jnp.exp(-v))
            return 0

        lax.fori_loop(0, GROUPS, sig_body, 0)
        pltpu.async_copy(outb.at[buf, pl.ds(0, CHUNK)],
                         out_hbm.at[pl.ds(out_base + c * CHUNK, CHUNK)],
                         osems.at[buf])

    start_gathers(0, 0)

    def pair_body(m, _):
        c0 = m * 2
        start_gathers(c0 + 1, 1)
        wait_gathers(c0, 0)
        compute_chunk(c0, 0)

        @pl.when(c0 + 2 < N_CHUNKS)
        def _():
            start_gathers(c0 + 2, 0)

        wait_gathers(c0 + 1, 1)
        compute_chunk(c0 + 1, 1)
        return 0

    lax.fori_loop(0, N_CHUNKS // 2, pair_body, 0)

    for buf, c in ((0, N_CHUNKS - 2), (1, N_CHUNKS - 1)):
        pltpu.make_async_copy(
            outb.at[buf, pl.ds(0, CHUNK)],
            out_hbm.at[pl.ds(out_base + c * CHUNK, CHUNK)],
            osems.at[buf]).wait()


@jax.jit
def _run(wi2d, ci2d, table, params):
    mesh = plsc.VectorSubcoreMesh(core_axis_name="c", subcore_axis_name="s")
    kern = pl.kernel(
        _sc_body,
        out_type=jax.ShapeDtypeStruct((TOTAL,), jnp.float32),
        mesh=mesh,
        scratch_types=[
            pltpu.VMEM((PER_W // IDXW, IDXW), jnp.int32),
            pltpu.VMEM((PER_W // IDXW, IDXW), jnp.int32),
            pltpu.VMEM((2, CHUNK, WPR), jnp.int32),
            pltpu.VMEM((2, CHUNK, WPR), jnp.int32),
            pltpu.VMEM((2, CHUNK + 16), jnp.float32),
            pltpu.VMEM((EMB_DIM + 16,), jnp.float32),
            pltpu.SemaphoreType.DMA((2,)),
            pltpu.SemaphoreType.DMA((2,)),
        ],
        compiler_params=pltpu.CompilerParams(
            needs_layout_passes=False, use_tc_tiling_on_sc=False),
    )
    return kern(wi2d, ci2d, table, params)


def kernel(word_ids, context_ids, table, fc_w, fc_b):
    packed = _transpose_pack(table.T.astype(jnp.float32))
    table_lin = packed.reshape(NROWS_LIN, WPR)

    def remap(ids):
        i = ids.T.reshape(TOTAL // IDXW, IDXW).astype(jnp.int32)
        return 4 * (i % SPLIT) + i // SPLIT

    wi2d = remap(word_ids)
    ci2d = remap(context_ids)
    params = jnp.concatenate(
        [fc_w.reshape(EMB_DIM), jnp.broadcast_to(fc_b, (16,))]).astype(jnp.float32)
    out = _run(wi2d, ci2d, table_lin, params)
    return out.reshape(L, B).T.reshape(B, L, 1)


# speedup vs baseline: 1.1058x; 1.0483x over previous
import jax
import jax.numpy as jnp
from jax import lax
from jax.experimental import pallas as pl
from jax.experimental.pallas import tpu as pltpu
from jax.experimental.pallas import tpu_sc as plsc

B = 16384
L = 50
EMB_DIM = 64
N_WORDS_TOTAL = 1000000
TOTAL = B * L
NW = 32
PER_W = TOTAL // NW
CHUNK = 512
IDXW = 128
ROWS_PER_CHUNK = CHUNK // IDXW
N_CHUNKS = PER_W // CHUNK
GROUPS = CHUNK // 16
UNROLL = 16
WPR = EMB_DIM // 2

BK = 2048
NBLK = 125
SPLIT = NBLK * BK
NROWS_LIN = 4 * SPLIT
IN_BLOCKS = (N_WORDS_TOTAL + BK - 1) // BK - 1


def _pack_bf16_pair(lo32, hi32):
    ul = lax.bitcast_convert_type(lo32, jnp.uint32)
    uh = lax.bitcast_convert_type(hi32, jnp.uint32)
    hm = jnp.uint32(0xFFFF0000)
    rl = (ul + jnp.uint32(0x7FFF) + ((ul >> 16) & jnp.uint32(1))) & hm
    rh = (uh + jnp.uint32(0x7FFF) + ((uh >> 16) & jnp.uint32(1))) & hm
    return lax.bitcast_convert_type(rl >> 16 | rh, jnp.int32)


def _tr_body(a_ref, b_ref, c_ref, d_ref, o_ref):
    parts = []
    for r in (a_ref, b_ref, c_ref, d_ref):
        v = r[...]
        w = _pack_bf16_pair(v[:WPR, :], v[WPR:, :])
        parts.append(w.T)
    o_ref[...] = jnp.concatenate(parts, axis=1)


def _transpose_pack(tt):
    def spec(j):
        return pl.BlockSpec(
            (EMB_DIM, BK),
            lambda k, j=j: (0, jnp.minimum(j * NBLK + k, IN_BLOCKS)))
    return pl.pallas_call(
        _tr_body,
        grid=(NBLK,),
        in_specs=[spec(0), spec(1), spec(2), spec(3)],
        out_specs=pl.BlockSpec((BK, 4 * WPR), lambda k: (k, 0)),
        out_shape=jax.ShapeDtypeStruct((SPLIT, 4 * WPR), jnp.int32),
    )(tt, tt, tt, tt)


def _sc_body(wi_hbm, ci_hbm, table_hbm, params_hbm, out_hbm,
             wi_v, ci_v, wr, cr, outb, pv, sems, osems):
    nc = 2
    wid = lax.axis_index("s") * nc + lax.axis_index("c")

    pltpu.sync_copy(params_hbm, pv)
    bias = pv[pl.ds(EMB_DIM, 16)]
    fw = [pv[pl.ds(k * 16, 16)] for k in range(EMB_DIM // 16)]
    mask15 = lax.iota(jnp.int32, 16) == 15
    himask = jnp.full((16,), -65536, jnp.int32)

    idx_row0 = wid * (PER_W // IDXW)
    out_base = wid * PER_W

    pltpu.sync_copy(wi_hbm.at[pl.ds(idx_row0, PER_W // IDXW)], wi_v)
    pltpu.sync_copy(ci_hbm.at[pl.ds(idx_row0, PER_W // IDXW)], ci_v)

    def start_gathers(c, buf):
        row = c * ROWS_PER_CHUNK
        for j in range(ROWS_PER_CHUNK):
            pltpu.async_copy(table_hbm.at[wi_v.at[row + j]],
                             wr.at[buf, pl.ds(j * IDXW, IDXW)], sems.at[buf])
            pltpu.async_copy(table_hbm.at[ci_v.at[row + j]],
                             cr.at[buf, pl.ds(j * IDXW, IDXW)], sems.at[buf])

    def wait_gathers(c, buf):
        row = c * ROWS_PER_CHUNK
        for j in range(ROWS_PER_CHUNK):
            pltpu.make_async_copy(table_hbm.at[wi_v.at[row + j]],
                                  wr.at[buf, pl.ds(j * IDXW, IDXW)],
                                  sems.at[buf]).wait()
            pltpu.make_async_copy(table_hbm.at[ci_v.at[row + j]],
                                  cr.at[buf, pl.ds(j * IDXW, IDXW)],
                                  sems.at[buf]).wait()

    def unpack(v):
        lo = plsc.bitcast(v << 16, jnp.float32)
        hi = plsc.bitcast(v & himask, jnp.float32)
        return lo, hi

    def compute_chunk(c, buf):
        @pl.when(c >= 2)
        def _():
            pltpu.make_async_copy(
                outb.at[buf, pl.ds(0, CHUNK)],
                out_hbm.at[pl.ds(out_base + (c - 2) * CHUNK, CHUNK)],
                osems.at[buf]).wait()

        def look_body(i, _):
            accs = []
            for u in range(UNROLL):
                ii = i * UNROLL + u
                w0lo, w0hi = unpack(wr[buf, ii, pl.ds(0, 16)])
                w1lo, w1hi = unpack(wr[buf, ii, pl.ds(16, 16)])
                c0lo, c0hi = unpack(cr[buf, ii, pl.ds(0, 16)])
                c1lo, c1hi = unpack(cr[buf, ii, pl.ds(16, 16)])
                acc = ((w0lo * c0lo) * fw[0] + (w1lo * c1lo) * fw[1]
                       + (w0hi * c0hi) * fw[2] + (w1hi * c1hi) * fw[3])
                accs.append(acc)
            cums = [plsc.cumsum(a) for a in accs]
            for u, cum in enumerate(cums):
                plsc.store_compressed(outb.at[buf, pl.ds(i * UNROLL + u, 16)],
                                      cum, mask=mask15)
            return 0

        lax.fori_loop(0, CHUNK // UNROLL, look_body, 0)

        def sig_body(g, _):
            v = outb[buf, pl.ds(g * 16, 16)] + bias
            outb[buf, pl.ds(g * 16, 16)] = 1.0 / (1.0 + jnp.exp(-v))
            return 0

        lax.fori_loop(0, GROUPS, sig_body, 0)
        pltpu.async_copy(outb.at[buf, pl.ds(0, CHUNK)],
                         out_hbm.at[pl.ds(out_base + c * CHUNK, CHUNK)],
                         osems.at[buf])

    start_gathers(0, 0)

    def pair_body(m, _):
        c0 = m * 2
        start_gathers(c0 + 1, 1)
        wait_gathers(c0, 0)
        compute_chunk(c0, 0)

        @pl.when(c0 + 2 < N_CHUNKS)
        def _():
            start_gathers(c0 + 2, 0)

        wait_gathers(c0 + 1, 1)
        compute_chunk(c0 + 1, 1)
        return 0

    lax.fori_loop(0, N_CHUNKS // 2, pair_body, 0)

    for buf, c in ((0, N_CHUNKS - 2), (1, N_CHUNKS - 1)):
        pltpu.make_async_copy(
            outb.at[buf, pl.ds(0, CHUNK)],
            out_hbm.at[pl.ds(out_base + c * CHUNK, CHUNK)],
            osems.at[buf]).wait()


@jax.jit
def _run(wi2d, ci2d, table, params):
    mesh = plsc.VectorSubcoreMesh(core_axis_name="c", subcore_axis_name="s")
    kern = pl.kernel(
        _sc_body,
        out_type=jax.ShapeDtypeStruct((TOTAL,), jnp.float32),
        mesh=mesh,
        scratch_types=[
            pltpu.VMEM((PER_W // IDXW, IDXW), jnp.int32),
            pltpu.VMEM((PER_W // IDXW, IDXW), jnp.int32),
            pltpu.VMEM((2, CHUNK, WPR), jnp.int32),
            pltpu.VMEM((2, CHUNK, WPR), jnp.int32),
            pltpu.VMEM((2, CHUNK + 16), jnp.float32),
            pltpu.VMEM((EMB_DIM + 16,), jnp.float32),
            pltpu.SemaphoreType.DMA((2,)),
            pltpu.SemaphoreType.DMA((2,)),
        ],
        compiler_params=pltpu.CompilerParams(
            needs_layout_passes=False, use_tc_tiling_on_sc=False),
    )
    return kern(wi2d, ci2d, table, params)


def kernel(word_ids, context_ids, table, fc_w, fc_b):
    packed = _transpose_pack(table.T.astype(jnp.float32))
    table_lin = packed.reshape(NROWS_LIN, WPR)

    def remap(ids):
        i = ids.T.reshape(TOTAL // IDXW, IDXW).astype(jnp.int32)
        return 4 * (i % SPLIT) + i // SPLIT

    wi2d = remap(word_ids)
    ci2d = remap(context_ids)
    params = jnp.concatenate(
        [fc_w.reshape(EMB_DIM), jnp.broadcast_to(fc_b, (16,))]).astype(jnp.float32)
    out = _run(wi2d, ci2d, table_lin, params)
    return out.reshape(L, B).T.reshape(B, L, 1)
